# baseline (speedup 1.0000x reference)
import functools

import jax
import jax.numpy as jnp
from jax import lax
from jax.experimental import pallas as pl
from jax.experimental.pallas import tpu as pltpu
from jax.experimental.pallas import tpu_sc as plsc

N, D, B, U, HS = 320000, 128, 16, 64, 256
EPS = 1e-5
L = 16
KD = D // L
NC, NS = 2, 16
NW = NC * NS
CHUNK_ROWS = 400
NCHUNK = 21
ROWS_PER_TILE = CHUNK_ROWS * NCHUNK
SC_ROWS = ROWS_PER_TILE * NW
TCBLK = 512
NTCBLK = (N - SC_ROWS) // TCBLK
TCBLK0 = SC_ROWS // TCBLK
GROUPS = CHUNK_ROWS // L

_NEG_INF = float("-inf")


def _pool_body(x_hbm, ids_hbm, omax_hbm, osum_hbm, ocnt_hbm,
               ids_v, xbuf0, xbuf1, accm, accs, accc, sem0, sem1):
    wid = lax.axis_index("c") * NS + lax.axis_index("s")
    base = wid * ROWS_PER_TILE

    def _chunk_src(c):
        return x_hbm.at[pl.ds((base + c * CHUNK_ROWS) * D, CHUNK_ROWS * D)]

    pltpu.async_copy(_chunk_src(0), xbuf0, sem0)

    neg = jnp.full((L,), _NEG_INF, jnp.float32)
    zero = jnp.zeros((L,), jnp.float32)
    for b in range(B):
        for k in range(KD):
            accm[b, pl.ds(k * L, L)] = neg
            accs[b, pl.ds(k * L, L)] = zero
    accc[...] = zero

    pltpu.sync_copy(ids_hbm.at[pl.ds(base, ROWS_PER_TILE)], ids_v)

    iota = lax.iota(jnp.int32, L)

    def _merge_rows(seg, m_regs, s_regs, nrows):
        for k in range(KD):
            sl = pl.ds(k * L, L)
            accm[seg, sl] = jnp.maximum(accm[seg, sl], m_regs[k])
            accs[seg, sl] = accs[seg, sl] + s_regs[k]
        accc[...] = accc[...] + jnp.where(iota == seg, nrows, 0.0)

    def _group(xbuf, c, g):
        row0 = c * CHUNK_ROWS + g * L
        ids = ids_v[pl.ds(row0, L)]
        s_first = ids[0]
        s_last = ids[L - 1]

        def uniform(_):
            lrow = g * L * D
            m_regs = [xbuf[pl.ds(lrow + k * L, L)] for k in range(KD)]
            s_regs = list(m_regs)
            for r in range(1, L):
                roff = lrow + r * D
                for k in range(KD):
                    v = xbuf[pl.ds(roff + k * L, L)]
                    m_regs[k] = jnp.maximum(m_regs[k], v)
                    s_regs[k] = s_regs[k] + v
            _merge_rows(s_first, m_regs, s_regs, jnp.float32(L))
            return 0

        def mixed(_):
            for r in range(L):
                seg = ids[r]
                roff = (g * L + r) * D
                regs = [xbuf[pl.ds(roff + k * L, L)] for k in range(KD)]
                _merge_rows(seg, regs, regs, jnp.float32(1))
            return 0

        lax.cond(s_first == s_last, uniform, mixed, 0)
        return g

    def _process(xbuf, c):
        cbase = c * CHUNK_ROWS
        first = ids_v[pl.ds(cbase, L)][0]
        last = ids_v[pl.ds(cbase + CHUNK_ROWS - L, L)][L - 1]

        def uniform_chunk(_):
            UN = 8
            m_regs = [xbuf[pl.ds(k * L, L)] for k in range(KD)]
            s_regs = list(m_regs)
            for r in range(1, UN):
                for k in range(KD):
                    v = xbuf[pl.ds(r * D + k * L, L)]
                    m_regs[k] = jnp.maximum(m_regs[k], v)
                    s_regs[k] = s_regs[k] + v

            def blk(bi, carry):
                m = list(carry[:KD])
                s = list(carry[KD:])
                boff = bi * (UN * D)
                for r in range(UN):
                    for k in range(KD):
                        v = xbuf[pl.ds(boff + r * D + k * L, L)]
                        m[k] = jnp.maximum(m[k], v)
                        s[k] = s[k] + v
                return tuple(m + s)

            carry = plsc.parallel_loop(
                1, CHUNK_ROWS // UN, carry=tuple(m_regs + s_regs))(
                    lambda bi, c: blk(bi, c))
            _merge_rows(first, carry[:KD], carry[KD:],
                        jnp.float32(CHUNK_ROWS))
            return 0

        def mixed_chunk(_):
            def g_body(g, _c):
                _group(xbuf, c, g)
                return _c
            lax.fori_loop(0, GROUPS, g_body, 0)
            return 0

        lax.cond(first == last, uniform_chunk, mixed_chunk, 0)

    def _wait(xbuf, sem):
        pltpu.make_async_copy(_chunk_src(0), xbuf, sem).wait()

    def pair_body(p, carry):
        c0 = p * 2
        pltpu.async_copy(_chunk_src(c0 + 1), xbuf1, sem1)
        _wait(xbuf0, sem0)
        _process(xbuf0, c0)
        pltpu.async_copy(_chunk_src(c0 + 2), xbuf0, sem0)
        _wait(xbuf1, sem1)
        _process(xbuf1, c0 + 1)
        return carry

    lax.fori_loop(0, (NCHUNK - 1) // 2, pair_body, 0)
    _wait(xbuf0, sem0)
    _process(xbuf0, NCHUNK - 1)

    pltpu.sync_copy(accm, omax_hbm.at[pl.ds(wid * B, B), :])
    pltpu.sync_copy(accs, osum_hbm.at[pl.ds(wid * B, B), :])
    pltpu.sync_copy(accc, ocnt_hbm.at[wid])


_pool = pl.kernel(
    _pool_body,
    out_type=(
        jax.ShapeDtypeStruct((NW * B, D), jnp.float32),
        jax.ShapeDtypeStruct((NW * B, D), jnp.float32),
        jax.ShapeDtypeStruct((NW, B), jnp.float32),
    ),
    mesh=plsc.VectorSubcoreMesh(
        core_axis_name="c", subcore_axis_name="s",
        num_cores=NC, num_subcores=NS),
    scratch_types=[
        pltpu.VMEM((ROWS_PER_TILE,), jnp.int32),
        pltpu.VMEM((CHUNK_ROWS * D,), jnp.float32),
        pltpu.VMEM((CHUNK_ROWS * D,), jnp.float32),
        pltpu.VMEM((B, D), jnp.float32),
        pltpu.VMEM((B, D), jnp.float32),
        pltpu.VMEM((B,), jnp.float32),
        pltpu.SemaphoreType.DMA,
        pltpu.SemaphoreType.DMA,
    ],
)


def _tcred_body(idc_ref, x_ref, tmax_ref, tsum_ref, tcnt_ref):
    i = pl.program_id(0)

    @pl.when(i == 0)
    def _init():
        tmax_ref[...] = jnp.full((B, D), _NEG_INF, jnp.float32)
        tsum_ref[...] = jnp.zeros((B, D), jnp.float32)
        tcnt_ref[...] = jnp.zeros((8, B), jnp.float32)

    xblk = x_ref[0]
    idc = idc_ref[...].reshape(TCBLK, 1)
    first = idc_ref[0, 0, 0]
    last = idc_ref[0, TCBLK - 1, 0]
    segio = lax.broadcasted_iota(jnp.int32, (B, D), 0)
    cntio = lax.broadcasted_iota(jnp.int32, (8, B), 1)

    def _acc(seg, bmax, bsum, nrows):
        m = tmax_ref[...]
        tmax_ref[...] = jnp.where(segio == seg,
                                  jnp.maximum(m, bmax[None, :]), m)
        s = tsum_ref[...]
        tsum_ref[...] = jnp.where(segio == seg, s + bsum[None, :], s)
        c = tcnt_ref[...]
        tcnt_ref[...] = jnp.where(cntio == seg, c + nrows, c)

    @pl.when(first == last)
    def _uniform():
        _acc(first, jnp.max(xblk, axis=0), jnp.sum(xblk, axis=0),
             jnp.float32(TCBLK))

    @pl.when(first != last)
    def _mixed():
        for s in range(B):
            mask = idc == s
            nrows = jnp.sum(mask.astype(jnp.float32))

            @pl.when(nrows > 0.0)
            def _(s=s, mask=mask, nrows=nrows):
                vm = jnp.where(mask, xblk, _NEG_INF)
                vs = jnp.where(mask, xblk, 0.0)
                _acc(s, jnp.max(vm, axis=0), jnp.sum(vs, axis=0), nrows)


_tcred = pl.pallas_call(
    _tcred_body,
    grid=(NTCBLK,),
    in_specs=[
        pl.BlockSpec((1, TCBLK, 1), lambda i: (i, 0, 0)),
        pl.BlockSpec((1, TCBLK, D), lambda i: (i + TCBLK0, 0, 0)),
    ],
    out_specs=[
        pl.BlockSpec((B, D), lambda i: (0, 0)),
        pl.BlockSpec((B, D), lambda i: (0, 0)),
        pl.BlockSpec((8, B), lambda i: (0, 0)),
    ],
    out_shape=(
        jax.ShapeDtypeStruct((B, D), jnp.float32),
        jax.ShapeDtypeStruct((B, D), jnp.float32),
        jax.ShapeDtypeStruct((8, B), jnp.float32),
    ),
)


def _mlp_body(u_ref, pmax_ref, psum_ref, pcnt_ref, tmax_ref, tsum_ref, tcnt_ref,
              W1t_ref, b1_ref, g1_ref, be1_ref,
              W2t_ref, b2_ref, g2_ref, be2_ref, o_ref):
    smax = jnp.maximum(jnp.max(pmax_ref[...], axis=0), tmax_ref[...])
    ssum = jnp.sum(psum_ref[...], axis=0) + tsum_ref[...]
    cnt = jnp.sum(pcnt_ref[...], axis=0) + tcnt_ref[0]
    smean = ssum / jnp.maximum(cnt, 1.0)[:, None]
    out = jnp.concatenate([u_ref[...], smax, smean, ssum], axis=1)
    h = lax.dot_general(out, W1t_ref[...], (((1,), (1,)), ((), ())),
                        preferred_element_type=jnp.float32) + b1_ref[...]
    h = (h / jnp.sqrt(1.0 + EPS)) * g1_ref[...] + be1_ref[...]
    h = jnp.maximum(h, 0.0)
    h = lax.dot_general(h, W2t_ref[...], (((1,), (1,)), ((), ())),
                        preferred_element_type=jnp.float32) + b2_ref[...]
    o_ref[...] = (h / jnp.sqrt(1.0 + EPS)) * g2_ref[...] + be2_ref[...]


_mlp = pl.pallas_call(
    _mlp_body,
    out_shape=jax.ShapeDtypeStruct((B, HS), jnp.float32),
)


@jax.jit
def kernel(x, u, batch, W1, b1, g1, be1, W2, b2, g2, be2):
    batch = batch.astype(jnp.int32)
    idc_tc = batch[SC_ROWS:].reshape(NTCBLK, TCBLK, 1)
    x3 = x.reshape(N // TCBLK, TCBLK, D)
    pmax, psum, pcnt = _pool(x.reshape(-1), batch)
    tmax, tsum, tcnt = _tcred(idc_tc, x3)
    return _mlp(u, pmax.reshape(NW, B, D), psum.reshape(NW, B, D), pcnt,
                tmax, tsum, tcnt,
                W1, b1, g1, be1, W2, b2, g2, be2)


# speedup vs baseline: 3.5951x; 3.5951x over previous
import functools

import jax
import jax.numpy as jnp
from jax import lax
from jax.experimental import pallas as pl
from jax.experimental.pallas import tpu as pltpu
from jax.experimental.pallas import tpu_sc as plsc

N, D, B, U, HS = 320000, 128, 16, 64, 256
EPS = 1e-5
L = 16
KD = D // L
NC, NS = 2, 16
NW = NC * NS
CHUNK_ROWS = 400
NCHUNK = 21
ROWS_PER_TILE = CHUNK_ROWS * NCHUNK
SC_ROWS = ROWS_PER_TILE * NW
TCBLK = 512
NTCBLK = (N - SC_ROWS) // TCBLK
TCBLK0 = SC_ROWS // TCBLK
GROUPS = CHUNK_ROWS // L

_NEG_INF = float("-inf")


def _pool_body(x_hbm, ids_hbm, omax_hbm, osum_hbm, ocnt_hbm,
               ids_v, xbuf0, xbuf1, accm, accs, accc, sem0, sem1):
    wid = lax.axis_index("c") * NS + lax.axis_index("s")
    base = wid * ROWS_PER_TILE

    def _chunk_src(c):
        return x_hbm.at[pl.ds(base + c * CHUNK_ROWS, CHUNK_ROWS), :]

    pltpu.async_copy(_chunk_src(0), xbuf0, sem0)

    neg = jnp.full((L,), _NEG_INF, jnp.float32)
    zero = jnp.zeros((L,), jnp.float32)
    for b in range(B):
        for k in range(KD):
            accm[b, pl.ds(k * L, L)] = neg
            accs[b, pl.ds(k * L, L)] = zero
    accc[...] = zero

    pltpu.sync_copy(ids_hbm.at[pl.ds(base, ROWS_PER_TILE)], ids_v)

    iota = lax.iota(jnp.int32, L)

    def _merge_rows(seg, m_regs, s_regs, nrows):
        for k in range(KD):
            sl = pl.ds(k * L, L)
            accm[seg, sl] = jnp.maximum(accm[seg, sl], m_regs[k])
            accs[seg, sl] = accs[seg, sl] + s_regs[k]
        accc[...] = accc[...] + jnp.where(iota == seg, nrows, 0.0)

    def _group(xbuf, c, g):
        row0 = c * CHUNK_ROWS + g * L
        ids = ids_v[pl.ds(row0, L)]
        s_first = ids[0]
        s_last = ids[L - 1]

        def uniform(_):
            lrow = g * L
            m_regs = [xbuf[lrow, pl.ds(k * L, L)] for k in range(KD)]
            s_regs = list(m_regs)
            for r in range(1, L):
                for k in range(KD):
                    v = xbuf[lrow + r, pl.ds(k * L, L)]
                    m_regs[k] = jnp.maximum(m_regs[k], v)
                    s_regs[k] = s_regs[k] + v
            _merge_rows(s_first, m_regs, s_regs, jnp.float32(L))
            return 0

        def mixed(_):
            for r in range(L):
                seg = ids[r]
                row = g * L + r
                regs = [xbuf[row, pl.ds(k * L, L)] for k in range(KD)]
                _merge_rows(seg, regs, regs, jnp.float32(1))
            return 0

        lax.cond(s_first == s_last, uniform, mixed, 0)
        return g

    def _process(xbuf, c):
        cbase = c * CHUNK_ROWS
        first = ids_v[pl.ds(cbase, L)][0]
        last = ids_v[pl.ds(cbase + CHUNK_ROWS - L, L)][L - 1]

        def uniform_chunk(_):
            UN = 8
            m_regs = [xbuf[0, pl.ds(k * L, L)] for k in range(KD)]
            s_regs = list(m_regs)
            for r in range(1, UN):
                for k in range(KD):
                    v = xbuf[r, pl.ds(k * L, L)]
                    m_regs[k] = jnp.maximum(m_regs[k], v)
                    s_regs[k] = s_regs[k] + v

            def blk(bi, carry):
                m = list(carry[:KD])
                s = list(carry[KD:])
                brow = bi * UN
                for r in range(UN):
                    for k in range(KD):
                        v = xbuf[brow + r, pl.ds(k * L, L)]
                        m[k] = jnp.maximum(m[k], v)
                        s[k] = s[k] + v
                return tuple(m + s)

            carry = plsc.parallel_loop(
                1, CHUNK_ROWS // UN, carry=tuple(m_regs + s_regs))(
                    lambda bi, c: blk(bi, c))
            _merge_rows(first, carry[:KD], carry[KD:],
                        jnp.float32(CHUNK_ROWS))
            return 0

        def mixed_chunk(_):
            def g_body(g, _c):
                _group(xbuf, c, g)
                return _c
            lax.fori_loop(0, GROUPS, g_body, 0)
            return 0

        lax.cond(first == last, uniform_chunk, mixed_chunk, 0)

    def _wait(xbuf, sem):
        pltpu.make_async_copy(_chunk_src(0), xbuf, sem).wait()

    def pair_body(p, carry):
        c0 = p * 2
        pltpu.async_copy(_chunk_src(c0 + 1), xbuf1, sem1)
        _wait(xbuf0, sem0)
        _process(xbuf0, c0)
        pltpu.async_copy(_chunk_src(c0 + 2), xbuf0, sem0)
        _wait(xbuf1, sem1)
        _process(xbuf1, c0 + 1)
        return carry

    lax.fori_loop(0, (NCHUNK - 1) // 2, pair_body, 0)
    _wait(xbuf0, sem0)
    _process(xbuf0, NCHUNK - 1)

    pltpu.sync_copy(accm, omax_hbm.at[pl.ds(wid * B, B), :])
    pltpu.sync_copy(accs, osum_hbm.at[pl.ds(wid * B, B), :])
    pltpu.sync_copy(accc, ocnt_hbm.at[wid])


_pool = pl.kernel(
    _pool_body,
    out_type=(
        jax.ShapeDtypeStruct((NW * B, D), jnp.float32),
        jax.ShapeDtypeStruct((NW * B, D), jnp.float32),
        jax.ShapeDtypeStruct((NW, B), jnp.float32),
    ),
    mesh=plsc.VectorSubcoreMesh(
        core_axis_name="c", subcore_axis_name="s",
        num_cores=NC, num_subcores=NS),
    scratch_types=[
        pltpu.VMEM((ROWS_PER_TILE,), jnp.int32),
        pltpu.VMEM((CHUNK_ROWS, D), jnp.float32),
        pltpu.VMEM((CHUNK_ROWS, D), jnp.float32),
        pltpu.VMEM((B, D), jnp.float32),
        pltpu.VMEM((B, D), jnp.float32),
        pltpu.VMEM((B,), jnp.float32),
        pltpu.SemaphoreType.DMA,
        pltpu.SemaphoreType.DMA,
    ],
)


def _tcred_body(idc_ref, x_ref, tmax_ref, tsum_ref, tcnt_ref):
    i = pl.program_id(0)

    @pl.when(i == 0)
    def _init():
        tmax_ref[...] = jnp.full((B, D), _NEG_INF, jnp.float32)
        tsum_ref[...] = jnp.zeros((B, D), jnp.float32)
        tcnt_ref[...] = jnp.zeros((8, B), jnp.float32)

    xblk = x_ref[...]
    ids = idc_ref[...].reshape(TCBLK)
    first = idc_ref[0, 0, 0]
    last = idc_ref[0, 0, TCBLK - 1]
    segio = lax.broadcasted_iota(jnp.int32, (B, D), 0)
    cntio = lax.broadcasted_iota(jnp.int32, (8, B), 1)

    def _acc(seg, bmax, bsum, nrows):
        m = tmax_ref[...]
        tmax_ref[...] = jnp.where(segio == seg,
                                  jnp.maximum(m, bmax[None, :]), m)
        s = tsum_ref[...]
        tsum_ref[...] = jnp.where(segio == seg, s + bsum[None, :], s)
        c = tcnt_ref[...]
        tcnt_ref[...] = jnp.where(cntio == seg, c + nrows, c)

    @pl.when(first == last)
    def _uniform():
        _acc(first, jnp.max(xblk, axis=0), jnp.sum(xblk, axis=0),
             jnp.float32(TCBLK))

    @pl.when(first != last)
    def _mixed():
        rowio = lax.broadcasted_iota(jnp.int32, (TCBLK, 1), 0)
        for s in range(B):
            lo = jnp.sum((ids < s).astype(jnp.int32))
            hi = jnp.sum((ids <= s).astype(jnp.int32))

            @pl.when(hi > lo)
            def _(s=s, lo=lo, hi=hi):
                mask = (rowio >= lo) & (rowio < hi)
                vm = jnp.where(mask, xblk, _NEG_INF)
                vs = jnp.where(mask, xblk, 0.0)
                _acc(s, jnp.max(vm, axis=0), jnp.sum(vs, axis=0),
                     (hi - lo).astype(jnp.float32))


_tcred = pl.pallas_call(
    _tcred_body,
    grid=(NTCBLK,),
    in_specs=[
        pl.BlockSpec((1, 1, TCBLK), lambda i: (i + TCBLK0, 0, 0)),
        pl.BlockSpec((TCBLK, D), lambda i: (i + TCBLK0, 0)),
    ],
    out_specs=[
        pl.BlockSpec((B, D), lambda i: (0, 0)),
        pl.BlockSpec((B, D), lambda i: (0, 0)),
        pl.BlockSpec((8, B), lambda i: (0, 0)),
    ],
    out_shape=(
        jax.ShapeDtypeStruct((B, D), jnp.float32),
        jax.ShapeDtypeStruct((B, D), jnp.float32),
        jax.ShapeDtypeStruct((8, B), jnp.float32),
    ),
)


def _mlp_body(u_ref, pmax_ref, psum_ref, pcnt_ref, tmax_ref, tsum_ref, tcnt_ref,
              W1t_ref, b1_ref, g1_ref, be1_ref,
              W2t_ref, b2_ref, g2_ref, be2_ref, o_ref):
    smax = jnp.maximum(jnp.max(pmax_ref[...], axis=0), tmax_ref[...])
    ssum = jnp.sum(psum_ref[...], axis=0) + tsum_ref[...]
    cnt = jnp.sum(pcnt_ref[...], axis=0) + tcnt_ref[0]
    smean = ssum / jnp.maximum(cnt, 1.0)[:, None]
    out = jnp.concatenate([u_ref[...], smax, smean, ssum], axis=1)
    h = lax.dot_general(out, W1t_ref[...], (((1,), (1,)), ((), ())),
                        preferred_element_type=jnp.float32) + b1_ref[...]
    h = (h / jnp.sqrt(1.0 + EPS)) * g1_ref[...] + be1_ref[...]
    h = jnp.maximum(h, 0.0)
    h = lax.dot_general(h, W2t_ref[...], (((1,), (1,)), ((), ())),
                        preferred_element_type=jnp.float32) + b2_ref[...]
    o_ref[...] = (h / jnp.sqrt(1.0 + EPS)) * g2_ref[...] + be2_ref[...]


_mlp = pl.pallas_call(
    _mlp_body,
    out_shape=jax.ShapeDtypeStruct((B, HS), jnp.float32),
)


@jax.jit
def kernel(x, u, batch, W1, b1, g1, be1, W2, b2, g2, be2):
    batch = batch.astype(jnp.int32)
    pmax, psum, pcnt = _pool(x, batch)
    tmax, tsum, tcnt = _tcred(batch.reshape(N // TCBLK, 1, TCBLK), x)
    return _mlp(u, pmax.reshape(NW, B, D), psum.reshape(NW, B, D), pcnt,
                tmax, tsum, tcnt,
                W1, b1, g1, be1, W2, b2, g2, be2)


# speedup vs baseline: 3.9805x; 1.1072x over previous
import functools

import jax
import jax.numpy as jnp
from jax import lax
from jax.experimental import pallas as pl
from jax.experimental.pallas import tpu as pltpu
from jax.experimental.pallas import tpu_sc as plsc

N, D, B, U, HS = 320000, 128, 16, 64, 256
EPS = 1e-5
L = 16
KD = D // L
NC, NS = 2, 16
NW = NC * NS
CHUNK_ROWS = 400
NCHUNK = 23
ROWS_PER_TILE = CHUNK_ROWS * NCHUNK
SC_ROWS = ROWS_PER_TILE * NW
TCBLK = 512
NTCBLK = (N - SC_ROWS) // TCBLK
TCBLK0 = SC_ROWS // TCBLK
GROUPS = CHUNK_ROWS // L

_NEG_INF = float("-inf")


def _pool_body(x_hbm, ids_hbm, omax_hbm, osum_hbm, ocnt_hbm,
               ids_v, xbuf0, xbuf1, accm, accs, accc, sem0, sem1):
    wid = lax.axis_index("c") * NS + lax.axis_index("s")
    base = wid * ROWS_PER_TILE

    def _chunk_src(c):
        return x_hbm.at[pl.ds(base + c * CHUNK_ROWS, CHUNK_ROWS), :]

    pltpu.async_copy(_chunk_src(0), xbuf0, sem0)

    neg = jnp.full((L,), _NEG_INF, jnp.float32)
    zero = jnp.zeros((L,), jnp.float32)
    for b in range(B):
        for k in range(KD):
            accm[b, pl.ds(k * L, L)] = neg
            accs[b, pl.ds(k * L, L)] = zero
    accc[...] = zero

    pltpu.sync_copy(ids_hbm.at[pl.ds(base, ROWS_PER_TILE)], ids_v)

    iota = lax.iota(jnp.int32, L)

    def _merge_rows(seg, m_regs, s_regs, nrows):
        for k in range(KD):
            sl = pl.ds(k * L, L)
            accm[seg, sl] = jnp.maximum(accm[seg, sl], m_regs[k])
            accs[seg, sl] = accs[seg, sl] + s_regs[k]
        accc[...] = accc[...] + jnp.where(iota == seg, nrows, 0.0)

    def _group(xbuf, c, g):
        row0 = c * CHUNK_ROWS + g * L
        ids = ids_v[pl.ds(row0, L)]
        s_first = ids[0]
        s_last = ids[L - 1]

        def uniform(_):
            lrow = g * L
            m_regs = [xbuf[lrow, pl.ds(k * L, L)] for k in range(KD)]
            s_regs = list(m_regs)
            for r in range(1, L):
                for k in range(KD):
                    v = xbuf[lrow + r, pl.ds(k * L, L)]
                    m_regs[k] = jnp.maximum(m_regs[k], v)
                    s_regs[k] = s_regs[k] + v
            _merge_rows(s_first, m_regs, s_regs, jnp.float32(L))
            return 0

        def mixed(_):
            for r in range(L):
                seg = ids[r]
                row = g * L + r
                regs = [xbuf[row, pl.ds(k * L, L)] for k in range(KD)]
                _merge_rows(seg, regs, regs, jnp.float32(1))
            return 0

        lax.cond(s_first == s_last, uniform, mixed, 0)
        return g

    def _process(xbuf, c):
        cbase = c * CHUNK_ROWS
        first = ids_v[pl.ds(cbase, L)][0]
        last = ids_v[pl.ds(cbase + CHUNK_ROWS - L, L)][L - 1]

        def uniform_chunk(_):
            UN = 8
            m_regs = [xbuf[0, pl.ds(k * L, L)] for k in range(KD)]
            s_regs = list(m_regs)
            for r in range(1, UN):
                for k in range(KD):
                    v = xbuf[r, pl.ds(k * L, L)]
                    m_regs[k] = jnp.maximum(m_regs[k], v)
                    s_regs[k] = s_regs[k] + v

            def blk(bi, carry):
                m = list(carry[:KD])
                s = list(carry[KD:])
                brow = bi * UN
                for r in range(UN):
                    for k in range(KD):
                        v = xbuf[brow + r, pl.ds(k * L, L)]
                        m[k] = jnp.maximum(m[k], v)
                        s[k] = s[k] + v
                return tuple(m + s)

            carry = plsc.parallel_loop(
                1, CHUNK_ROWS // UN, carry=tuple(m_regs + s_regs))(
                    lambda bi, c: blk(bi, c))
            _merge_rows(first, carry[:KD], carry[KD:],
                        jnp.float32(CHUNK_ROWS))
            return 0

        def mixed_chunk(_):
            def g_body(g, _c):
                _group(xbuf, c, g)
                return _c
            lax.fori_loop(0, GROUPS, g_body, 0)
            return 0

        lax.cond(first == last, uniform_chunk, mixed_chunk, 0)

    def _wait(xbuf, sem):
        pltpu.make_async_copy(_chunk_src(0), xbuf, sem).wait()

    def pair_body(p, carry):
        c0 = p * 2
        pltpu.async_copy(_chunk_src(c0 + 1), xbuf1, sem1)
        _wait(xbuf0, sem0)
        _process(xbuf0, c0)
        pltpu.async_copy(_chunk_src(c0 + 2), xbuf0, sem0)
        _wait(xbuf1, sem1)
        _process(xbuf1, c0 + 1)
        return carry

    lax.fori_loop(0, (NCHUNK - 1) // 2, pair_body, 0)
    _wait(xbuf0, sem0)
    _process(xbuf0, NCHUNK - 1)

    pltpu.sync_copy(accm, omax_hbm.at[pl.ds(wid * B, B), :])
    pltpu.sync_copy(accs, osum_hbm.at[pl.ds(wid * B, B), :])
    pltpu.sync_copy(accc, ocnt_hbm.at[wid])


_pool = pl.kernel(
    _pool_body,
    out_type=(
        jax.ShapeDtypeStruct((NW * B, D), jnp.float32),
        jax.ShapeDtypeStruct((NW * B, D), jnp.float32),
        jax.ShapeDtypeStruct((NW, B), jnp.float32),
    ),
    mesh=plsc.VectorSubcoreMesh(
        core_axis_name="c", subcore_axis_name="s",
        num_cores=NC, num_subcores=NS),
    scratch_types=[
        pltpu.VMEM((ROWS_PER_TILE,), jnp.int32),
        pltpu.VMEM((CHUNK_ROWS, D), jnp.float32),
        pltpu.VMEM((CHUNK_ROWS, D), jnp.float32),
        pltpu.VMEM((B, D), jnp.float32),
        pltpu.VMEM((B, D), jnp.float32),
        pltpu.VMEM((B,), jnp.float32),
        pltpu.SemaphoreType.DMA,
        pltpu.SemaphoreType.DMA,
    ],
)


def _tcred_body(idc_ref, x_ref, tmax_ref, tsum_ref, tcnt_ref):
    i = pl.program_id(0)

    @pl.when(i == 0)
    def _init():
        tmax_ref[...] = jnp.full((B, D), _NEG_INF, jnp.float32)
        tsum_ref[...] = jnp.zeros((B, D), jnp.float32)
        tcnt_ref[...] = jnp.zeros((8, B), jnp.float32)

    xblk = x_ref[...]
    ids = idc_ref[...].reshape(TCBLK)
    first = idc_ref[0, 0, 0]
    last = idc_ref[0, 0, TCBLK - 1]
    segio = lax.broadcasted_iota(jnp.int32, (B, D), 0)
    cntio = lax.broadcasted_iota(jnp.int32, (8, B), 1)

    def _acc(seg, bmax, bsum, nrows):
        m = tmax_ref[...]
        tmax_ref[...] = jnp.where(segio == seg,
                                  jnp.maximum(m, bmax[None, :]), m)
        s = tsum_ref[...]
        tsum_ref[...] = jnp.where(segio == seg, s + bsum[None, :], s)
        c = tcnt_ref[...]
        tcnt_ref[...] = jnp.where(cntio == seg, c + nrows, c)

    @pl.when(first == last)
    def _uniform():
        _acc(first, jnp.max(xblk, axis=0), jnp.sum(xblk, axis=0),
             jnp.float32(TCBLK))

    @pl.when(first != last)
    def _mixed():
        rowio = lax.broadcasted_iota(jnp.int32, (TCBLK, 1), 0)
        for s in range(B):
            lo = jnp.sum((ids < s).astype(jnp.int32))
            hi = jnp.sum((ids <= s).astype(jnp.int32))

            @pl.when(hi > lo)
            def _(s=s, lo=lo, hi=hi):
                mask = (rowio >= lo) & (rowio < hi)
                vm = jnp.where(mask, xblk, _NEG_INF)
                vs = jnp.where(mask, xblk, 0.0)
                _acc(s, jnp.max(vm, axis=0), jnp.sum(vs, axis=0),
                     (hi - lo).astype(jnp.float32))


_tcred = pl.pallas_call(
    _tcred_body,
    grid=(NTCBLK,),
    in_specs=[
        pl.BlockSpec((1, 1, TCBLK), lambda i: (i + TCBLK0, 0, 0)),
        pl.BlockSpec((TCBLK, D), lambda i: (i + TCBLK0, 0)),
    ],
    out_specs=[
        pl.BlockSpec((B, D), lambda i: (0, 0)),
        pl.BlockSpec((B, D), lambda i: (0, 0)),
        pl.BlockSpec((8, B), lambda i: (0, 0)),
    ],
    out_shape=(
        jax.ShapeDtypeStruct((B, D), jnp.float32),
        jax.ShapeDtypeStruct((B, D), jnp.float32),
        jax.ShapeDtypeStruct((8, B), jnp.float32),
    ),
)


def _mlp_body(u_ref, pmax_ref, psum_ref, pcnt_ref, tmax_ref, tsum_ref, tcnt_ref,
              W1t_ref, b1_ref, g1_ref, be1_ref,
              W2t_ref, b2_ref, g2_ref, be2_ref, o_ref):
    smax = jnp.maximum(jnp.max(pmax_ref[...], axis=0), tmax_ref[...])
    ssum = jnp.sum(psum_ref[...], axis=0) + tsum_ref[...]
    cnt = jnp.sum(pcnt_ref[...], axis=0) + tcnt_ref[0]
    smean = ssum / jnp.maximum(cnt, 1.0)[:, None]
    out = jnp.concatenate([u_ref[...], smax, smean, ssum], axis=1)
    h = lax.dot_general(out, W1t_ref[...], (((1,), (1,)), ((), ())),
                        preferred_element_type=jnp.float32) + b1_ref[...]
    h = (h / jnp.sqrt(1.0 + EPS)) * g1_ref[...] + be1_ref[...]
    h = jnp.maximum(h, 0.0)
    h = lax.dot_general(h, W2t_ref[...], (((1,), (1,)), ((), ())),
                        preferred_element_type=jnp.float32) + b2_ref[...]
    o_ref[...] = (h / jnp.sqrt(1.0 + EPS)) * g2_ref[...] + be2_ref[...]


_mlp = pl.pallas_call(
    _mlp_body,
    out_shape=jax.ShapeDtypeStruct((B, HS), jnp.float32),
)


@jax.jit
def kernel(x, u, batch, W1, b1, g1, be1, W2, b2, g2, be2):
    batch = batch.astype(jnp.int32)
    pmax, psum, pcnt = _pool(x, batch)
    tmax, tsum, tcnt = _tcred(batch.reshape(N // TCBLK, 1, TCBLK), x)
    return _mlp(u, pmax.reshape(NW, B, D), psum.reshape(NW, B, D), pcnt,
                tmax, tsum, tcnt,
                W1, b1, g1, be1, W2, b2, g2, be2)


# speedup vs baseline: 4.1486x; 1.0422x over previous
import functools

import jax
import jax.numpy as jnp
from jax import lax
from jax.experimental import pallas as pl
from jax.experimental.pallas import tpu as pltpu
from jax.experimental.pallas import tpu_sc as plsc

N, D, B, U, HS = 320000, 128, 16, 64, 256
EPS = 1e-5
L = 16
KD = D // L
NC, NS = 2, 16
NW = NC * NS
CHUNK_ROWS = 400
NCHUNK = 22
ROWS_PER_TILE = CHUNK_ROWS * NCHUNK
SC_ROWS = ROWS_PER_TILE * NW
TCBLK = 512
NTCBLK = (N - SC_ROWS) // TCBLK
TCBLK0 = SC_ROWS // TCBLK
GROUPS = CHUNK_ROWS // L

_NEG_INF = float("-inf")


def _pool_body(x_hbm, ids_hbm, omax_hbm, osum_hbm, ocnt_hbm,
               ids_v, xbuf0, xbuf1, accm, accs, accc, sem0, sem1):
    wid = lax.axis_index("c") * NS + lax.axis_index("s")
    base = wid * ROWS_PER_TILE

    def _chunk_src(c):
        return x_hbm.at[pl.ds(base + c * CHUNK_ROWS, CHUNK_ROWS), :]

    pltpu.async_copy(_chunk_src(0), xbuf0, sem0)

    neg = jnp.full((L,), _NEG_INF, jnp.float32)
    zero = jnp.zeros((L,), jnp.float32)
    for b in range(B):
        for k in range(KD):
            accm[b, pl.ds(k * L, L)] = neg
            accs[b, pl.ds(k * L, L)] = zero
    accc[...] = zero

    pltpu.sync_copy(ids_hbm.at[pl.ds(base, ROWS_PER_TILE)], ids_v)

    iota = lax.iota(jnp.int32, L)

    def _merge_rows(seg, m_regs, s_regs, nrows):
        for k in range(KD):
            sl = pl.ds(k * L, L)
            accm[seg, sl] = jnp.maximum(accm[seg, sl], m_regs[k])
            accs[seg, sl] = accs[seg, sl] + s_regs[k]
        accc[...] = accc[...] + jnp.where(iota == seg, nrows, 0.0)

    def _group(xbuf, c, g):
        row0 = c * CHUNK_ROWS + g * L
        ids = ids_v[pl.ds(row0, L)]
        s_first = ids[0]
        s_last = ids[L - 1]

        def uniform(_):
            lrow = g * L
            m_regs = [xbuf[lrow, pl.ds(k * L, L)] for k in range(KD)]
            s_regs = list(m_regs)
            for r in range(1, L):
                for k in range(KD):
                    v = xbuf[lrow + r, pl.ds(k * L, L)]
                    m_regs[k] = jnp.maximum(m_regs[k], v)
                    s_regs[k] = s_regs[k] + v
            _merge_rows(s_first, m_regs, s_regs, jnp.float32(L))
            return 0

        def mixed(_):
            for r in range(L):
                seg = ids[r]
                row = g * L + r
                regs = [xbuf[row, pl.ds(k * L, L)] for k in range(KD)]
                _merge_rows(seg, regs, regs, jnp.float32(1))
            return 0

        lax.cond(s_first == s_last, uniform, mixed, 0)
        return g

    def _process(xbuf, c):
        cbase = c * CHUNK_ROWS
        first = ids_v[pl.ds(cbase, L)][0]
        last = ids_v[pl.ds(cbase + CHUNK_ROWS - L, L)][L - 1]

        def uniform_chunk(_):
            UN = 8
            m_regs = [xbuf[0, pl.ds(k * L, L)] for k in range(KD)]
            s_regs = list(m_regs)
            for r in range(1, UN):
                for k in range(KD):
                    v = xbuf[r, pl.ds(k * L, L)]
                    m_regs[k] = jnp.maximum(m_regs[k], v)
                    s_regs[k] = s_regs[k] + v

            def blk(bi, carry):
                m = list(carry[:KD])
                s = list(carry[KD:])
                brow = bi * UN
                for r in range(UN):
                    for k in range(KD):
                        v = xbuf[brow + r, pl.ds(k * L, L)]
                        m[k] = jnp.maximum(m[k], v)
                        s[k] = s[k] + v
                return tuple(m + s)

            carry = plsc.parallel_loop(
                1, CHUNK_ROWS // UN, carry=tuple(m_regs + s_regs))(
                    lambda bi, c: blk(bi, c))
            _merge_rows(first, carry[:KD], carry[KD:],
                        jnp.float32(CHUNK_ROWS))
            return 0

        def mixed_chunk(_):
            def g_body(g, _c):
                _group(xbuf, c, g)
                return _c
            lax.fori_loop(0, GROUPS, g_body, 0)
            return 0

        lax.cond(first == last, uniform_chunk, mixed_chunk, 0)

    def _wait(xbuf, sem):
        pltpu.make_async_copy(_chunk_src(0), xbuf, sem).wait()

    def pair_body(p, carry):
        c0 = p * 2
        pltpu.async_copy(_chunk_src(c0 + 1), xbuf1, sem1)
        _wait(xbuf0, sem0)
        _process(xbuf0, c0)

        @pl.when(c0 + 2 < NCHUNK)
        def _():
            pltpu.async_copy(_chunk_src(c0 + 2), xbuf0, sem0)

        _wait(xbuf1, sem1)
        _process(xbuf1, c0 + 1)
        return carry

    lax.fori_loop(0, NCHUNK // 2, pair_body, 0)
    if NCHUNK % 2:
        _wait(xbuf0, sem0)
        _process(xbuf0, NCHUNK - 1)

    pltpu.sync_copy(accm, omax_hbm.at[pl.ds(wid * B, B), :])
    pltpu.sync_copy(accs, osum_hbm.at[pl.ds(wid * B, B), :])
    pltpu.sync_copy(accc, ocnt_hbm.at[wid])


_pool = pl.kernel(
    _pool_body,
    out_type=(
        jax.ShapeDtypeStruct((NW * B, D), jnp.float32),
        jax.ShapeDtypeStruct((NW * B, D), jnp.float32),
        jax.ShapeDtypeStruct((NW, B), jnp.float32),
    ),
    mesh=plsc.VectorSubcoreMesh(
        core_axis_name="c", subcore_axis_name="s",
        num_cores=NC, num_subcores=NS),
    scratch_types=[
        pltpu.VMEM((ROWS_PER_TILE,), jnp.int32),
        pltpu.VMEM((CHUNK_ROWS, D), jnp.float32),
        pltpu.VMEM((CHUNK_ROWS, D), jnp.float32),
        pltpu.VMEM((B, D), jnp.float32),
        pltpu.VMEM((B, D), jnp.float32),
        pltpu.VMEM((B,), jnp.float32),
        pltpu.SemaphoreType.DMA,
        pltpu.SemaphoreType.DMA,
    ],
)


def _tcred_body(idc_ref, x_ref, tmax_ref, tsum_ref, tcnt_ref):
    i = pl.program_id(0)

    @pl.when(i == 0)
    def _init():
        tmax_ref[...] = jnp.full((B, D), _NEG_INF, jnp.float32)
        tsum_ref[...] = jnp.zeros((B, D), jnp.float32)
        tcnt_ref[...] = jnp.zeros((8, B), jnp.float32)

    xblk = x_ref[...]
    ids = idc_ref[...].reshape(TCBLK)
    first = idc_ref[0, 0, 0]
    last = idc_ref[0, 0, TCBLK - 1]
    segio = lax.broadcasted_iota(jnp.int32, (B, D), 0)
    cntio = lax.broadcasted_iota(jnp.int32, (8, B), 1)

    def _acc(seg, bmax, bsum, nrows):
        m = tmax_ref[...]
        tmax_ref[...] = jnp.where(segio == seg,
                                  jnp.maximum(m, bmax[None, :]), m)
        s = tsum_ref[...]
        tsum_ref[...] = jnp.where(segio == seg, s + bsum[None, :], s)
        c = tcnt_ref[...]
        tcnt_ref[...] = jnp.where(cntio == seg, c + nrows, c)

    @pl.when(first == last)
    def _uniform():
        _acc(first, jnp.max(xblk, axis=0), jnp.sum(xblk, axis=0),
             jnp.float32(TCBLK))

    @pl.when(first != last)
    def _mixed():
        rowio = lax.broadcasted_iota(jnp.int32, (TCBLK, 1), 0)
        for s in range(B):
            lo = jnp.sum((ids < s).astype(jnp.int32))
            hi = jnp.sum((ids <= s).astype(jnp.int32))

            @pl.when(hi > lo)
            def _(s=s, lo=lo, hi=hi):
                mask = (rowio >= lo) & (rowio < hi)
                vm = jnp.where(mask, xblk, _NEG_INF)
                vs = jnp.where(mask, xblk, 0.0)
                _acc(s, jnp.max(vm, axis=0), jnp.sum(vs, axis=0),
                     (hi - lo).astype(jnp.float32))


_tcred = pl.pallas_call(
    _tcred_body,
    grid=(NTCBLK,),
    in_specs=[
        pl.BlockSpec((1, 1, TCBLK), lambda i: (i + TCBLK0, 0, 0)),
        pl.BlockSpec((TCBLK, D), lambda i: (i + TCBLK0, 0)),
    ],
    out_specs=[
        pl.BlockSpec((B, D), lambda i: (0, 0)),
        pl.BlockSpec((B, D), lambda i: (0, 0)),
        pl.BlockSpec((8, B), lambda i: (0, 0)),
    ],
    out_shape=(
        jax.ShapeDtypeStruct((B, D), jnp.float32),
        jax.ShapeDtypeStruct((B, D), jnp.float32),
        jax.ShapeDtypeStruct((8, B), jnp.float32),
    ),
)


def _mlp_body(u_ref, pmax_ref, psum_ref, pcnt_ref, tmax_ref, tsum_ref, tcnt_ref,
              W1t_ref, b1_ref, g1_ref, be1_ref,
              W2t_ref, b2_ref, g2_ref, be2_ref, o_ref):
    smax = jnp.maximum(jnp.max(pmax_ref[...], axis=0), tmax_ref[...])
    ssum = jnp.sum(psum_ref[...], axis=0) + tsum_ref[...]
    cnt = jnp.sum(pcnt_ref[...], axis=0) + tcnt_ref[0]
    smean = ssum / jnp.maximum(cnt, 1.0)[:, None]
    out = jnp.concatenate([u_ref[...], smax, smean, ssum], axis=1)
    h = lax.dot_general(out, W1t_ref[...], (((1,), (1,)), ((), ())),
                        preferred_element_type=jnp.float32) + b1_ref[...]
    h = (h / jnp.sqrt(1.0 + EPS)) * g1_ref[...] + be1_ref[...]
    h = jnp.maximum(h, 0.0)
    h = lax.dot_general(h, W2t_ref[...], (((1,), (1,)), ((), ())),
                        preferred_element_type=jnp.float32) + b2_ref[...]
    o_ref[...] = (h / jnp.sqrt(1.0 + EPS)) * g2_ref[...] + be2_ref[...]


_mlp = pl.pallas_call(
    _mlp_body,
    out_shape=jax.ShapeDtypeStruct((B, HS), jnp.float32),
)


@jax.jit
def kernel(x, u, batch, W1, b1, g1, be1, W2, b2, g2, be2):
    batch = batch.astype(jnp.int32)
    pmax, psum, pcnt = _pool(x, batch)
    tmax, tsum, tcnt = _tcred(batch.reshape(N // TCBLK, 1, TCBLK), x)
    return _mlp(u, pmax.reshape(NW, B, D), psum.reshape(NW, B, D), pcnt,
                tmax, tsum, tcnt,
                W1, b1, g1, be1, W2, b2, g2, be2)
